# Initial kernel scaffold; baseline (speedup 1.0000x reference)
import jax
import jax.numpy as jnp
from jax.experimental import pallas as pl


def kernel(pos, x, batch, edge_index, W1, W2, W3, W4, W5, W6):
    raise NotImplementedError("write your pallas kernel here")


# speedup vs baseline: 4.4052x; 4.4052x over previous
import functools

import jax
import jax.numpy as jnp
from jax import lax
from jax.experimental import pallas as pl
from jax.experimental.pallas import tpu as pltpu
from jax.experimental.pallas import tpu_sc as plsc

N_NODES = 50000
N_EDGES = 800000
HID = 64
NP = 50048
NC, NS = 2, 16
NW = NC * NS
CH = 128
J = 200
JB = 25
EPT = CH * J
EPAD = EPT * NW
BR = 2176
GR = NP // BR


@functools.lru_cache(None)
def _make_apply(C):
    mesh = plsc.VectorSubcoreMesh(core_axis_name="c", subcore_axis_name="s")

    nzt = NP // CH

    @functools.partial(
        pl.kernel,
        out_type=jax.ShapeDtypeStruct((NC, NP, C), jnp.float32),
        mesh=mesh,
        scratch_types=[
            pltpu.VMEM((JB, CH), jnp.int32),
            pltpu.VMEM((JB, CH), jnp.int32),
            pltpu.VMEM((CH, C), jnp.float32),
            pltpu.VMEM((CH, C), jnp.float32),
            pltpu.VMEM_SHARED((NP, C), jnp.float32),
            pltpu.SemaphoreType.DMA,
        ],
        compiler_params=pltpu.CompilerParams(use_tc_tiling_on_sc=False),
    )
    def apply_k(tab, gidx, sidx, ztile, out, gv, sv, rows, zbuf, acc, sem):
        cid = lax.axis_index("c")
        sid = lax.axis_index("s")
        wid = sid * NC + cid

        pltpu.sync_copy(ztile, zbuf)

        def zero_body(i, carry):
            t = i * NS + sid

            @pl.when(t < nzt)
            def _():
                pltpu.sync_copy(zbuf, acc.at[pl.ds(t * CH, CH)])

            return carry

        lax.fori_loop(0, (nzt + NS - 1) // NS, zero_body, 0, unroll=False)
        plsc.subcore_barrier()

        def blk(b, carry):
            pltpu.sync_copy(gidx.at[wid].at[pl.ds(b * JB, JB)], gv)
            pltpu.sync_copy(sidx.at[wid].at[pl.ds(b * JB, JB)], sv)

            def body(j, c2):
                pltpu.async_copy(tab.at[gv.at[j]], rows, sem).wait()
                pltpu.sync_copy(rows, acc.at[sv.at[j]], add=True)
                return c2

            lax.fori_loop(0, JB, body, 0, unroll=False)
            return carry

        lax.fori_loop(0, J // JB, blk, 0, unroll=False)
        plsc.subcore_barrier()

        @pl.when(sid == 0)
        def _dump():
            pltpu.sync_copy(acc, out.at[cid])

    return apply_k


def _prep(degp, x0p):
    def body(deg_ref, x0_ref, dinv_ref, yt_ref):
        deg = deg_ref[0, :, 0:1] + deg_ref[1, :, 0:1]
        dinv = jnp.where(deg > 0, lax.rsqrt(deg), 0.0)
        dinv_ref[...] = dinv
        yt_ref[...] = x0_ref[...] * dinv

    return pl.pallas_call(
        body,
        grid=(GR,),
        in_specs=[
            pl.BlockSpec((NC, BR, 32), lambda i: (0, i, 0)),
            pl.BlockSpec((BR, 32), lambda i: (i, 0)),
        ],
        out_specs=[
            pl.BlockSpec((BR, 1), lambda i: (i, 0)),
            pl.BlockSpec((BR, 32), lambda i: (i, 0)),
        ],
        out_shape=[
            jax.ShapeDtypeStruct((NP, 1), jnp.float32),
            jax.ShapeDtypeStruct((NP, 32), jnp.float32),
        ],
    )(degp, x0p)


def _cheb_step(partials, dinv, prev2):
    nch = len(partials)
    first = prev2 is None
    Cs = [p.shape[-1] for p in partials]

    def body(*refs):
        prefs = refs[:nch]
        dref = refs[nch]
        p2refs = () if first else refs[nch + 1: nch + 1 + nch]
        orefs = refs[nch + 1 + (0 if first else nch):]
        txrefs = orefs[:nch]
        ytrefs = orefs[nch:]
        dv = dref[...]
        for q in range(nch):
            ltx = -(prefs[q][0] + prefs[q][1]) * dv
            tx = ltx if first else 2.0 * ltx - p2refs[q][...]
            txrefs[q][...] = tx
            ytrefs[q][...] = tx * dv

    in_specs = [pl.BlockSpec((NC, BR, C), lambda i: (0, i, 0)) for C in Cs]
    in_specs.append(pl.BlockSpec((BR, 1), lambda i: (i, 0)))
    args = list(partials) + [dinv]
    if not first:
        in_specs += [pl.BlockSpec((BR, C), lambda i: (i, 0)) for C in Cs]
        args += list(prev2)
    out_specs = [pl.BlockSpec((BR, C), lambda i: (i, 0)) for C in Cs] * 2
    out_shape = [jax.ShapeDtypeStruct((NP, C), jnp.float32) for C in Cs] * 2

    outs = pl.pallas_call(
        body, grid=(GR,), in_specs=in_specs, out_specs=out_specs,
        out_shape=out_shape,
    )(*args)
    return list(outs[:nch]), list(outs[nch:])


def _layer_mm(tx_list, W, dinv):
    K = W.shape[0]
    nch = len(tx_list[0])
    Cs = [t.shape[-1] for t in tx_list[0]]
    F = W.shape[1]

    def body(*refs):
        nin = K * nch
        xrefs = refs[:nin]
        wref = refs[nin]
        dref = refs[nin + 1]
        oA, oB, yA, yB = refs[nin + 2: nin + 6]
        acc = jnp.zeros((BR, HID), jnp.float32)
        for k in range(K):
            off = 0
            for q in range(nch):
                xb = xrefs[k * nch + q][...]
                acc = acc + jnp.dot(
                    xb, wref[k, off:off + Cs[q], :],
                    preferred_element_type=jnp.float32)
                off += Cs[q]
        o = jnp.where(acc > 0, acc, 0.5 * acc)
        dv = dref[...]
        oA[...] = o[:, :32]
        oB[...] = o[:, 32:]
        yA[...] = o[:, :32] * dv
        yB[...] = o[:, 32:] * dv

    in_specs = [pl.BlockSpec((BR, C), lambda i: (i, 0))
                for _k in range(K) for C in Cs]
    in_specs.append(pl.BlockSpec((K, F, HID), lambda i: (0, 0, 0)))
    in_specs.append(pl.BlockSpec((BR, 1), lambda i: (i, 0)))
    out_specs = [pl.BlockSpec((BR, 32), lambda i: (i, 0))] * 4
    out_shape = [jax.ShapeDtypeStruct((NP, 32), jnp.float32)] * 4
    args = [t for txs in tx_list for t in txs] + [W, dinv]
    o = pl.pallas_call(
        body, grid=(GR,), in_specs=in_specs, out_specs=out_specs,
        out_shape=out_shape,
    )(*args)
    return [o[0], o[1]], [o[2], o[3]]


def _final(outs, x0p):
    def body(*refs):
        (a1, b1, a2, b2, a3, b3, a4, b4, a5, b5, a6, b6, xref, gref) = refs
        i = pl.program_id(0)

        @pl.when(i == 0)
        def _init():
            gref[...] = jnp.zeros_like(gref)

        xb = xref[...]
        pairs = [(a1, a4), (b1, b4), (a2, a5), (b2, b5), (a3, a6), (b3, b6)]
        accs = []
        for (a, b) in pairs:
            h = a[...] + b[...]
            accs.append(lax.dot_general(
                h, xb, (((0,), (0,)), ((), ())),
                preferred_element_type=jnp.float32))
        g = gref[...] + jnp.concatenate(accs, axis=0)

        @pl.when(i < GR - 1)
        def _store():
            gref[...] = g

        @pl.when(i == GR - 1)
        def _done():
            gref[...] = g * lax.rsqrt(jnp.sum(g * g, axis=1, keepdims=True))

    in_specs = [pl.BlockSpec((BR, 32), lambda i: (i, 0))] * 12
    in_specs.append(pl.BlockSpec((BR, 32), lambda i: (i, 0)))
    return pl.pallas_call(
        body, grid=(GR,), in_specs=in_specs,
        out_specs=pl.BlockSpec((192, 32), lambda i: (0, 0)),
        out_shape=jax.ShapeDtypeStruct((192, 32), jnp.float32),
    )(*outs, x0p)


def _cheb_layer(x_chunks, yt_chunks, W, dinv, gidx, sidx, ztile):
    K = W.shape[0]
    Txs = [list(x_chunks)]
    yt = list(yt_chunks)
    for k in range(1, K):
        partials = [_make_apply(t.shape[-1])(t, gidx, sidx, ztile)
                    for t in yt]
        prev2 = None if k == 1 else Txs[k - 2]
        tx, yt = _cheb_step(partials, dinv, prev2)
        Txs.append(tx)
    return _layer_mm(Txs, W, dinv)


def kernel(pos, x, batch, edge_index, W1, W2, W3, W4, W5, W6):
    src = edge_index[0].astype(jnp.int32)
    dst = edge_index[1].astype(jnp.int32)

    srcg = jnp.where(src == dst, N_NODES, src)
    padg = jnp.full((EPAD - N_EDGES,), N_NODES, jnp.int32)
    pads = jnp.zeros((EPAD - N_EDGES,), jnp.int32)
    gidx = jnp.concatenate([srcg, padg]).reshape(NW, J, CH)
    sidx_dst = jnp.concatenate([dst, pads]).reshape(NW, J, CH)
    sidx_src = jnp.concatenate([src, pads]).reshape(NW, J, CH)

    x0 = jnp.concatenate([pos, x], axis=1).astype(jnp.float32)
    x0p = jnp.pad(x0, ((0, NP - N_NODES), (0, 32 - 9)))
    e0 = jnp.pad(jnp.ones((N_NODES, 1), jnp.float32),
                 ((0, NP - N_NODES), (0, 31)))
    ztile = jnp.zeros((CH, 32), jnp.float32)
    W1p = jnp.pad(W1, ((0, 0), (0, 23), (0, 0)))
    W4p = jnp.pad(W4, ((0, 0), (0, 23), (0, 0)))

    degp = _make_apply(32)(e0, gidx, sidx_src, ztile)
    dinv, ytab0 = _prep(degp, x0p)

    run = functools.partial(_cheb_layer, dinv=dinv, gidx=gidx,
                            sidx=sidx_dst, ztile=ztile)
    out1, yt1 = run([x0p], [ytab0], W1p)
    out2, yt2 = run(out1, yt1, W2)
    out3, _ = run(out2, yt2, W3)
    out4, yt4 = run([x0p], [ytab0], W4p)
    out5, yt5 = run(out4, yt4, W5)
    out6, _ = run(out5, yt5, W6)

    G = _final(out1 + out2 + out3 + out4 + out5 + out6, x0p)
    return G[:, :9].reshape(1, 192, 9)


# speedup vs baseline: 5.0655x; 1.1499x over previous
import functools

import jax
import jax.numpy as jnp
from jax import lax
from jax.experimental import pallas as pl
from jax.experimental.pallas import tpu as pltpu
from jax.experimental.pallas import tpu_sc as plsc

N_NODES = 50000
N_EDGES = 800000
HID = 64
NP = 50048
NC, NS = 2, 16
NW = NC * NS
CH = 128
J = 200
JB = 40
NBUF = 4
EPT = CH * J
EPAD = EPT * NW
BR = 2176
GR = NP // BR


@functools.lru_cache(None)
def _make_apply(C):
    mesh = plsc.VectorSubcoreMesh(core_axis_name="c", subcore_axis_name="s")

    nzt = NP // CH

    @functools.partial(
        pl.kernel,
        out_type=jax.ShapeDtypeStruct((NC, NP, C), jnp.float32),
        mesh=mesh,
        scratch_types=[
            pltpu.VMEM((JB, CH), jnp.int32),
            pltpu.VMEM((JB, CH), jnp.int32),
        ] + [pltpu.VMEM((CH, C), jnp.float32)] * NBUF + [
            pltpu.VMEM_SHARED((NP, C), jnp.float32),
        ] + [pltpu.SemaphoreType.DMA] * NBUF,
        compiler_params=pltpu.CompilerParams(use_tc_tiling_on_sc=False),
    )
    def apply_k(tab, gidx, sidx, ztile, out, gv, sv, r0, r1, r2, r3,
                acc, s0, s1, s2, s3):
        rows = (r0, r1, r2, r3)
        sems = (s0, s1, s2, s3)
        cid = lax.axis_index("c")
        sid = lax.axis_index("s")
        wid = sid * NC + cid

        def zero_body(i, carry):
            t = i * NS + sid

            @pl.when(t < nzt)
            def _():
                pltpu.sync_copy(ztile, acc.at[pl.ds(t * CH, CH)])

            return carry

        lax.fori_loop(0, (nzt + NS - 1) // NS, zero_body, 0, unroll=False)
        plsc.subcore_barrier()

        def blk(b, carry):
            pltpu.sync_copy(gidx.at[wid].at[pl.ds(b * JB, JB)], gv)
            pltpu.sync_copy(sidx.at[wid].at[pl.ds(b * JB, JB)], sv)
            for q in range(NBUF):
                pltpu.async_copy(tab.at[gv.at[q]], rows[q], sems[q])

            def body(i, c2):
                base = i * NBUF
                for q in range(NBUF):
                    j = base + q
                    pltpu.make_async_copy(
                        tab.at[gv.at[j]], rows[q], sems[q]).wait()
                    pltpu.sync_copy(rows[q], acc.at[sv.at[j]], add=True)
                    jn = j + NBUF

                    @pl.when(jn < JB)
                    def _fire(q=q, jn=jn):
                        pltpu.async_copy(tab.at[gv.at[jn]], rows[q], sems[q])

                return c2

            lax.fori_loop(0, JB // NBUF, body, 0, unroll=False)
            return carry

        lax.fori_loop(0, J // JB, blk, 0, unroll=False)
        plsc.subcore_barrier()

        @pl.when(sid == 0)
        def _dump():
            pltpu.sync_copy(acc, out.at[cid])

    return apply_k


def _prep(degp, x0p):
    def body(deg_ref, x0_ref, dinv_ref, yt_ref):
        deg = deg_ref[0, :, 0:1] + deg_ref[1, :, 0:1]
        dinv = jnp.where(deg > 0, lax.rsqrt(deg), 0.0)
        dinv_ref[...] = dinv
        yt_ref[...] = x0_ref[...] * dinv

    return pl.pallas_call(
        body,
        grid=(GR,),
        in_specs=[
            pl.BlockSpec((NC, BR, 32), lambda i: (0, i, 0)),
            pl.BlockSpec((BR, 32), lambda i: (i, 0)),
        ],
        out_specs=[
            pl.BlockSpec((BR, 1), lambda i: (i, 0)),
            pl.BlockSpec((BR, 32), lambda i: (i, 0)),
        ],
        out_shape=[
            jax.ShapeDtypeStruct((NP, 1), jnp.float32),
            jax.ShapeDtypeStruct((NP, 32), jnp.float32),
        ],
    )(degp, x0p)


def _cheb_step(partials, dinv, prev2):
    nch = len(partials)
    first = prev2 is None
    Cs = [p.shape[-1] for p in partials]

    def body(*refs):
        prefs = refs[:nch]
        dref = refs[nch]
        p2refs = () if first else refs[nch + 1: nch + 1 + nch]
        orefs = refs[nch + 1 + (0 if first else nch):]
        txrefs = orefs[:nch]
        ytrefs = orefs[nch:]
        dv = dref[...]
        for q in range(nch):
            ltx = -(prefs[q][0] + prefs[q][1]) * dv
            tx = ltx if first else 2.0 * ltx - p2refs[q][...]
            txrefs[q][...] = tx
            ytrefs[q][...] = tx * dv

    in_specs = [pl.BlockSpec((NC, BR, C), lambda i: (0, i, 0)) for C in Cs]
    in_specs.append(pl.BlockSpec((BR, 1), lambda i: (i, 0)))
    args = list(partials) + [dinv]
    if not first:
        in_specs += [pl.BlockSpec((BR, C), lambda i: (i, 0)) for C in Cs]
        args += list(prev2)
    out_specs = [pl.BlockSpec((BR, C), lambda i: (i, 0)) for C in Cs] * 2
    out_shape = [jax.ShapeDtypeStruct((NP, C), jnp.float32) for C in Cs] * 2

    outs = pl.pallas_call(
        body, grid=(GR,), in_specs=in_specs, out_specs=out_specs,
        out_shape=out_shape,
    )(*args)
    return list(outs[:nch]), list(outs[nch:])


def _layer_mm(tx_list, W, dinv):
    K = W.shape[0]
    nch = len(tx_list[0])
    Cs = [t.shape[-1] for t in tx_list[0]]
    F = W.shape[1]

    def body(*refs):
        nin = K * nch
        xrefs = refs[:nin]
        wref = refs[nin]
        dref = refs[nin + 1]
        oA, oB, yA, yB = refs[nin + 2: nin + 6]
        acc = jnp.zeros((BR, HID), jnp.float32)
        for k in range(K):
            off = 0
            for q in range(nch):
                xb = xrefs[k * nch + q][...]
                acc = acc + jnp.dot(
                    xb, wref[k, off:off + Cs[q], :],
                    preferred_element_type=jnp.float32)
                off += Cs[q]
        o = jnp.where(acc > 0, acc, 0.5 * acc)
        dv = dref[...]
        oA[...] = o[:, :32]
        oB[...] = o[:, 32:]
        yA[...] = o[:, :32] * dv
        yB[...] = o[:, 32:] * dv

    in_specs = [pl.BlockSpec((BR, C), lambda i: (i, 0))
                for _k in range(K) for C in Cs]
    in_specs.append(pl.BlockSpec((K, F, HID), lambda i: (0, 0, 0)))
    in_specs.append(pl.BlockSpec((BR, 1), lambda i: (i, 0)))
    out_specs = [pl.BlockSpec((BR, 32), lambda i: (i, 0))] * 4
    out_shape = [jax.ShapeDtypeStruct((NP, 32), jnp.float32)] * 4
    args = [t for txs in tx_list for t in txs] + [W, dinv]
    o = pl.pallas_call(
        body, grid=(GR,), in_specs=in_specs, out_specs=out_specs,
        out_shape=out_shape,
    )(*args)
    return [o[0], o[1]], [o[2], o[3]]


def _final(outs, x0p):
    def body(*refs):
        (a1, b1, a2, b2, a3, b3, a4, b4, a5, b5, a6, b6, xref, gref) = refs
        i = pl.program_id(0)

        @pl.when(i == 0)
        def _init():
            gref[...] = jnp.zeros_like(gref)

        xb = xref[...]
        pairs = [(a1, a4), (b1, b4), (a2, a5), (b2, b5), (a3, a6), (b3, b6)]
        accs = []
        for (a, b) in pairs:
            h = a[...] + b[...]
            accs.append(lax.dot_general(
                h, xb, (((0,), (0,)), ((), ())),
                preferred_element_type=jnp.float32))
        g = gref[...] + jnp.concatenate(accs, axis=0)

        @pl.when(i < GR - 1)
        def _store():
            gref[...] = g

        @pl.when(i == GR - 1)
        def _done():
            gref[...] = g * lax.rsqrt(jnp.sum(g * g, axis=1, keepdims=True))

    in_specs = [pl.BlockSpec((BR, 32), lambda i: (i, 0))] * 12
    in_specs.append(pl.BlockSpec((BR, 32), lambda i: (i, 0)))
    return pl.pallas_call(
        body, grid=(GR,), in_specs=in_specs,
        out_specs=pl.BlockSpec((192, 32), lambda i: (0, 0)),
        out_shape=jax.ShapeDtypeStruct((192, 32), jnp.float32),
    )(*outs, x0p)


def _cheb_layer(x_chunks, yt_chunks, W, dinv, gidx, sidx, ztile):
    K = W.shape[0]
    Txs = [list(x_chunks)]
    yt = list(yt_chunks)
    for k in range(1, K):
        partials = [_make_apply(t.shape[-1])(t, gidx, sidx, ztile)
                    for t in yt]
        prev2 = None if k == 1 else Txs[k - 2]
        tx, yt = _cheb_step(partials, dinv, prev2)
        Txs.append(tx)
    return _layer_mm(Txs, W, dinv)


def kernel(pos, x, batch, edge_index, W1, W2, W3, W4, W5, W6):
    src = edge_index[0].astype(jnp.int32)
    dst = edge_index[1].astype(jnp.int32)

    srcg = jnp.where(src == dst, N_NODES, src)
    padg = jnp.full((EPAD - N_EDGES,), N_NODES, jnp.int32)
    pads = jnp.zeros((EPAD - N_EDGES,), jnp.int32)
    gidx = jnp.concatenate([srcg, padg]).reshape(NW, J, CH)
    sidx_dst = jnp.concatenate([dst, pads]).reshape(NW, J, CH)
    sidx_src = jnp.concatenate([src, pads]).reshape(NW, J, CH)

    x0 = jnp.concatenate([pos, x], axis=1).astype(jnp.float32)
    x0p = jnp.pad(x0, ((0, NP - N_NODES), (0, 32 - 9)))
    e0 = jnp.pad(jnp.ones((N_NODES, 1), jnp.float32),
                 ((0, NP - N_NODES), (0, 31)))
    ztile = jnp.zeros((CH, 32), jnp.float32)
    W1p = jnp.pad(W1, ((0, 0), (0, 23), (0, 0)))
    W4p = jnp.pad(W4, ((0, 0), (0, 23), (0, 0)))

    degp = _make_apply(32)(e0, gidx, sidx_src, ztile)
    dinv, ytab0 = _prep(degp, x0p)

    run = functools.partial(_cheb_layer, dinv=dinv, gidx=gidx,
                            sidx=sidx_dst, ztile=ztile)
    out1, yt1 = run([x0p], [ytab0], W1p)
    out2, yt2 = run(out1, yt1, W2)
    out3, _ = run(out2, yt2, W3)
    out4, yt4 = run([x0p], [ytab0], W4p)
    out5, yt5 = run(out4, yt4, W5)
    out6, _ = run(out5, yt5, W6)

    G = _final(out1 + out2 + out3 + out4 + out5 + out6, x0p)
    return G[:, :9].reshape(1, 192, 9)


# speedup vs baseline: 5.0694x; 1.0008x over previous
import functools

import jax
import jax.numpy as jnp
from jax import lax
from jax.experimental import pallas as pl
from jax.experimental.pallas import tpu as pltpu
from jax.experimental.pallas import tpu_sc as plsc

N_NODES = 50000
N_EDGES = 800000
HID = 64
NP = 50048
NC, NS = 2, 16
NW = NC * NS
CH = 128
J = 200
JB = 40
NBUF = 5
PF = 3
EPT = CH * J
EPAD = EPT * NW
BR = 2176
GR = NP // BR


@functools.lru_cache(None)
def _make_apply(C):
    mesh = plsc.VectorSubcoreMesh(core_axis_name="c", subcore_axis_name="s")

    nzt = NP // CH

    @functools.partial(
        pl.kernel,
        out_type=jax.ShapeDtypeStruct((NC, NP, C), jnp.float32),
        mesh=mesh,
        scratch_types=[
            pltpu.VMEM((JB, CH), jnp.int32),
            pltpu.VMEM((JB, CH), jnp.int32),
        ] + [pltpu.VMEM((CH, C), jnp.float32)] * NBUF + [
            pltpu.VMEM_SHARED((NP, C), jnp.float32),
        ] + [pltpu.SemaphoreType.DMA] * (2 * NBUF),
        compiler_params=pltpu.CompilerParams(use_tc_tiling_on_sc=False),
    )
    def apply_k(tab, gidx, sidx, ztile, out, gv, sv, r0, r1, r2, r3, r4,
                acc, g0, g1, g2, g3, g4, t0, t1, t2, t3, t4):
        rows = (r0, r1, r2, r3, r4)
        gsem = (g0, g1, g2, g3, g4)
        ssem = (t0, t1, t2, t3, t4)
        cid = lax.axis_index("c")
        sid = lax.axis_index("s")
        wid = sid * NC + cid

        def zero_body(i, carry):
            t = i * NS + sid

            @pl.when(t < nzt)
            def _():
                pltpu.sync_copy(ztile, acc.at[pl.ds(t * CH, CH)])

            return carry

        lax.fori_loop(0, (nzt + NS - 1) // NS, zero_body, 0, unroll=False)
        plsc.subcore_barrier()

        def blk(b, carry):
            pltpu.sync_copy(gidx.at[wid].at[pl.ds(b * JB, JB)], gv)
            pltpu.sync_copy(sidx.at[wid].at[pl.ds(b * JB, JB)], sv)
            for q in range(PF):
                pltpu.async_copy(tab.at[gv.at[q]], rows[q], gsem[q])

            def body(i, c2):
                base = i * NBUF
                for q in range(NBUF):
                    j = base + q
                    qf = (q + PF) % NBUF
                    jf = j + PF
                    jw = jf - NBUF

                    @pl.when((jf < JB) & (jw >= 0))
                    def _drain(qf=qf, jw=jw):
                        pltpu.make_async_copy(
                            rows[qf], acc.at[sv.at[jw]], ssem[qf]).wait()

                    @pl.when(jf < JB)
                    def _fire(qf=qf, jf=jf):
                        pltpu.async_copy(tab.at[gv.at[jf]], rows[qf],
                                         gsem[qf])

                    pltpu.make_async_copy(
                        tab.at[gv.at[j]], rows[q], gsem[q]).wait()
                    pltpu.async_copy(rows[q], acc.at[sv.at[j]], ssem[q],
                                     add=True)

                return c2

            lax.fori_loop(0, JB // NBUF, body, 0, unroll=False)
            for q in range(NBUF):
                pltpu.make_async_copy(
                    rows[q], acc.at[sv.at[JB - NBUF + q]], ssem[q]).wait()
            return carry

        lax.fori_loop(0, J // JB, blk, 0, unroll=False)
        plsc.subcore_barrier()

        @pl.when(sid == 0)
        def _dump():
            pltpu.sync_copy(acc, out.at[cid])

    return apply_k


def _prep(degp, x0p):
    def body(deg_ref, x0_ref, dinv_ref, yt_ref):
        deg = deg_ref[0, :, 0:1] + deg_ref[1, :, 0:1]
        dinv = jnp.where(deg > 0, lax.rsqrt(deg), 0.0)
        dinv_ref[...] = dinv
        yt_ref[...] = x0_ref[...] * dinv

    return pl.pallas_call(
        body,
        grid=(GR,),
        in_specs=[
            pl.BlockSpec((NC, BR, 32), lambda i: (0, i, 0)),
            pl.BlockSpec((BR, 32), lambda i: (i, 0)),
        ],
        out_specs=[
            pl.BlockSpec((BR, 1), lambda i: (i, 0)),
            pl.BlockSpec((BR, 32), lambda i: (i, 0)),
        ],
        out_shape=[
            jax.ShapeDtypeStruct((NP, 1), jnp.float32),
            jax.ShapeDtypeStruct((NP, 32), jnp.float32),
        ],
    )(degp, x0p)


def _cheb_step(partials, dinv, prev2):
    nch = len(partials)
    first = prev2 is None
    Cs = [p.shape[-1] for p in partials]

    def body(*refs):
        prefs = refs[:nch]
        dref = refs[nch]
        p2refs = () if first else refs[nch + 1: nch + 1 + nch]
        orefs = refs[nch + 1 + (0 if first else nch):]
        txrefs = orefs[:nch]
        ytrefs = orefs[nch:]
        dv = dref[...]
        for q in range(nch):
            ltx = -(prefs[q][0] + prefs[q][1]) * dv
            tx = ltx if first else 2.0 * ltx - p2refs[q][...]
            txrefs[q][...] = tx
            ytrefs[q][...] = tx * dv

    in_specs = [pl.BlockSpec((NC, BR, C), lambda i: (0, i, 0)) for C in Cs]
    in_specs.append(pl.BlockSpec((BR, 1), lambda i: (i, 0)))
    args = list(partials) + [dinv]
    if not first:
        in_specs += [pl.BlockSpec((BR, C), lambda i: (i, 0)) for C in Cs]
        args += list(prev2)
    out_specs = [pl.BlockSpec((BR, C), lambda i: (i, 0)) for C in Cs] * 2
    out_shape = [jax.ShapeDtypeStruct((NP, C), jnp.float32) for C in Cs] * 2

    outs = pl.pallas_call(
        body, grid=(GR,), in_specs=in_specs, out_specs=out_specs,
        out_shape=out_shape,
    )(*args)
    return list(outs[:nch]), list(outs[nch:])


def _layer_mm(tx_list, W, dinv):
    K = W.shape[0]
    nch = len(tx_list[0])
    Cs = [t.shape[-1] for t in tx_list[0]]
    F = W.shape[1]

    def body(*refs):
        nin = K * nch
        xrefs = refs[:nin]
        wref = refs[nin]
        dref = refs[nin + 1]
        oA, oB, yA, yB = refs[nin + 2: nin + 6]
        acc = jnp.zeros((BR, HID), jnp.float32)
        for k in range(K):
            off = 0
            for q in range(nch):
                xb = xrefs[k * nch + q][...]
                acc = acc + jnp.dot(
                    xb, wref[k, off:off + Cs[q], :],
                    preferred_element_type=jnp.float32)
                off += Cs[q]
        o = jnp.where(acc > 0, acc, 0.5 * acc)
        dv = dref[...]
        oA[...] = o[:, :32]
        oB[...] = o[:, 32:]
        yA[...] = o[:, :32] * dv
        yB[...] = o[:, 32:] * dv

    in_specs = [pl.BlockSpec((BR, C), lambda i: (i, 0))
                for _k in range(K) for C in Cs]
    in_specs.append(pl.BlockSpec((K, F, HID), lambda i: (0, 0, 0)))
    in_specs.append(pl.BlockSpec((BR, 1), lambda i: (i, 0)))
    out_specs = [pl.BlockSpec((BR, 32), lambda i: (i, 0))] * 4
    out_shape = [jax.ShapeDtypeStruct((NP, 32), jnp.float32)] * 4
    args = [t for txs in tx_list for t in txs] + [W, dinv]
    o = pl.pallas_call(
        body, grid=(GR,), in_specs=in_specs, out_specs=out_specs,
        out_shape=out_shape,
    )(*args)
    return [o[0], o[1]], [o[2], o[3]]


def _final(outs, x0p):
    def body(*refs):
        (a1, b1, a2, b2, a3, b3, a4, b4, a5, b5, a6, b6, xref, gref) = refs
        i = pl.program_id(0)

        @pl.when(i == 0)
        def _init():
            gref[...] = jnp.zeros_like(gref)

        xb = xref[...]
        pairs = [(a1, a4), (b1, b4), (a2, a5), (b2, b5), (a3, a6), (b3, b6)]
        accs = []
        for (a, b) in pairs:
            h = a[...] + b[...]
            accs.append(lax.dot_general(
                h, xb, (((0,), (0,)), ((), ())),
                preferred_element_type=jnp.float32))
        g = gref[...] + jnp.concatenate(accs, axis=0)

        @pl.when(i < GR - 1)
        def _store():
            gref[...] = g

        @pl.when(i == GR - 1)
        def _done():
            gref[...] = g * lax.rsqrt(jnp.sum(g * g, axis=1, keepdims=True))

    in_specs = [pl.BlockSpec((BR, 32), lambda i: (i, 0))] * 12
    in_specs.append(pl.BlockSpec((BR, 32), lambda i: (i, 0)))
    return pl.pallas_call(
        body, grid=(GR,), in_specs=in_specs,
        out_specs=pl.BlockSpec((192, 32), lambda i: (0, 0)),
        out_shape=jax.ShapeDtypeStruct((192, 32), jnp.float32),
    )(*outs, x0p)


def _cheb_layer(x_chunks, yt_chunks, W, dinv, gidx, sidx, ztile):
    K = W.shape[0]
    Txs = [list(x_chunks)]
    yt = list(yt_chunks)
    for k in range(1, K):
        partials = [_make_apply(t.shape[-1])(t, gidx, sidx, ztile)
                    for t in yt]
        prev2 = None if k == 1 else Txs[k - 2]
        tx, yt = _cheb_step(partials, dinv, prev2)
        Txs.append(tx)
    return _layer_mm(Txs, W, dinv)


def kernel(pos, x, batch, edge_index, W1, W2, W3, W4, W5, W6):
    src = edge_index[0].astype(jnp.int32)
    dst = edge_index[1].astype(jnp.int32)

    srcg = jnp.where(src == dst, N_NODES, src)
    padg = jnp.full((EPAD - N_EDGES,), N_NODES, jnp.int32)
    pads = jnp.zeros((EPAD - N_EDGES,), jnp.int32)
    gidx = jnp.concatenate([srcg, padg]).reshape(NW, J, CH)
    sidx_dst = jnp.concatenate([dst, pads]).reshape(NW, J, CH)
    sidx_src = jnp.concatenate([src, pads]).reshape(NW, J, CH)

    x0 = jnp.concatenate([pos, x], axis=1).astype(jnp.float32)
    x0p = jnp.pad(x0, ((0, NP - N_NODES), (0, 32 - 9)))
    e0 = jnp.pad(jnp.ones((N_NODES, 1), jnp.float32),
                 ((0, NP - N_NODES), (0, 31)))
    ztile = jnp.zeros((CH, 32), jnp.float32)
    W1p = jnp.pad(W1, ((0, 0), (0, 23), (0, 0)))
    W4p = jnp.pad(W4, ((0, 0), (0, 23), (0, 0)))

    degp = _make_apply(32)(e0, gidx, sidx_src, ztile)
    dinv, ytab0 = _prep(degp, x0p)

    run = functools.partial(_cheb_layer, dinv=dinv, gidx=gidx,
                            sidx=sidx_dst, ztile=ztile)
    out1, yt1 = run([x0p], [ytab0], W1p)
    out2, yt2 = run(out1, yt1, W2)
    out3, _ = run(out2, yt2, W3)
    out4, yt4 = run([x0p], [ytab0], W4p)
    out5, yt5 = run(out4, yt4, W5)
    out6, _ = run(out5, yt5, W6)

    G = _final(out1 + out2 + out3 + out4 + out5 + out6, x0p)
    return G[:, :9].reshape(1, 192, 9)
